# Initial kernel scaffold; baseline (speedup 1.0000x reference)
import jax
import jax.numpy as jnp
from jax.experimental import pallas as pl


def kernel(nodes, x, neigh, W_homo, W_att, b_att, q_att):
    raise NotImplementedError("write your pallas kernel here")


# speedup vs baseline: 2.7790x; 2.7790x over previous
import functools

import jax
import jax.numpy as jnp
from jax import lax
from jax.experimental import pallas as pl
from jax.experimental.pallas import tpu as pltpu
from jax.experimental.pallas import tpu_sc as plsc

N_NODES_C = 10000
D = 512
DEG = 16
N_META = 3
E = 512
B = 4096
A_DIM = 128

NC = 2
NS = 16
NW = NC * NS
BPW = B // NW


NB_W = 128


def _sc_gather_agg(nodes, x, nb_all):
    mesh = plsc.VectorSubcoreMesh(
        core_axis_name="c", subcore_axis_name="s", num_cores=NC, num_subcores=NS
    )

    @functools.partial(
        pl.kernel,
        out_type=[
            jax.ShapeDtypeStruct((B, D), jnp.float32),
            jax.ShapeDtypeStruct((N_META, B, D), jnp.float32),
        ],
        mesh=mesh,
        scratch_types=[
            pltpu.VMEM((BPW,), jnp.int32),
            pltpu.VMEM((BPW, NB_W), jnp.int32),
            pltpu.VMEM((BPW, D), jnp.float32),
            pltpu.VMEM((DEG, D), jnp.float32),
            pltpu.VMEM((DEG, D), jnp.float32),
            pltpu.SemaphoreType.DMA,
            pltpu.SemaphoreType.DMA,
            pltpu.SemaphoreType.DMA,
        ],
    )
    def sc_kernel(nodes_h, x_h, nb_h, self_o, agg_o,
                  idx_v, nb_v, buf, rows0, rows1, semi, sem0, sem1):
        wid = lax.axis_index("s") * NC + lax.axis_index("c")
        base = wid * BPW

        pltpu.sync_copy(nodes_h.at[pl.ds(base, BPW)], idx_v)
        pltpu.async_copy(nb_h.at[idx_v], nb_v, semi)
        pltpu.async_copy(x_h.at[idx_v], buf, sem0).wait()
        pltpu.sync_copy(buf, self_o.at[pl.ds(base, BPW)])
        pltpu.make_async_copy(nb_h.at[pl.ds(0, BPW)], nb_v, semi).wait()

        inv_deg = jnp.float32(1.0 / DEG)

        def reduce_rows(rows, n):
            def col_body(c, carry):
                sl = pl.ds(c * 16, 16)
                s = rows[0, sl]
                for r in range(1, DEG):
                    s = s + rows[r, sl]
                buf[n, sl] = s * inv_deg
                return carry
            lax.fori_loop(0, D // 16, col_body, 0)

        for m in range(N_META):
            co = m * DEG
            pltpu.async_copy(x_h.at[nb_v.at[0, pl.ds(co, DEG)]], rows0, sem0)

            def node_pair(i2, carry):
                n0i = 2 * i2
                n1i = n0i + 1
                pltpu.async_copy(x_h.at[nb_v.at[n1i, pl.ds(co, DEG)]], rows1, sem1)
                pltpu.make_async_copy(x_h.at[pl.ds(0, DEG)], rows0, sem0).wait()
                reduce_rows(rows0, n0i)

                @pl.when(n1i + 1 < BPW)
                def _():
                    pltpu.async_copy(
                        x_h.at[nb_v.at[n1i + 1, pl.ds(co, DEG)]], rows0, sem0)

                pltpu.make_async_copy(x_h.at[pl.ds(0, DEG)], rows1, sem1).wait()
                reduce_rows(rows1, n1i)
                return carry

            lax.fori_loop(0, BPW // 2, node_pair, 0)
            pltpu.sync_copy(buf, agg_o.at[m].at[pl.ds(base, BPW)])

    return sc_kernel(nodes, x, nb_all)


TB = 512
NT = B // TB


def _tc_encode(self_feat, agg, w_top, w_bot, w_att, b_att2, q_att2):
    def body(self_r, agg_r, wt_r, wb_r, wa_r, ba_r, qa_r, h_r, s_r):
        sf = self_r[...]
        lane = lax.broadcasted_iota(jnp.int32, (1, A_DIM), 1)
        s_vec = jnp.zeros((1, A_DIM), jnp.float32)
        for m in range(N_META):
            h = jnp.maximum(
                jnp.dot(sf, wt_r[m], preferred_element_type=jnp.float32)
                + jnp.dot(agg_r[m], wb_r[m], preferred_element_type=jnp.float32),
                0.0,
            )
            h_r[m] = h
            t = jnp.tanh(
                jnp.dot(h, wa_r[...], preferred_element_type=jnp.float32) + ba_r[...]
            )
            sm = jnp.sum(jnp.dot(t, qa_r[...], preferred_element_type=jnp.float32))
            s_vec = s_vec + jnp.where(lane == m, sm, 0.0)
        s_r[pl.ds(pl.program_id(0), 1), :] = s_vec

    return pl.pallas_call(
        body,
        grid=(NT,),
        in_specs=[
            pl.BlockSpec((TB, D), lambda i: (i, 0)),
            pl.BlockSpec((N_META, TB, D), lambda i: (0, i, 0)),
            pl.BlockSpec((N_META, D, E), lambda i: (0, 0, 0)),
            pl.BlockSpec((N_META, D, E), lambda i: (0, 0, 0)),
            pl.BlockSpec((E, A_DIM), lambda i: (0, 0)),
            pl.BlockSpec((1, A_DIM), lambda i: (0, 0)),
            pl.BlockSpec((A_DIM, 1), lambda i: (0, 0)),
        ],
        out_specs=[
            pl.BlockSpec((N_META, TB, E), lambda i: (0, i, 0)),
            pl.BlockSpec((NT, A_DIM), lambda i: (0, 0)),
        ],
        out_shape=[
            jax.ShapeDtypeStruct((N_META, B, E), jnp.float32),
            jax.ShapeDtypeStruct((NT, A_DIM), jnp.float32),
        ],
    )(self_feat, agg, w_top, w_bot, w_att, b_att2, q_att2)


def _tc_combine(s_part, h_all):
    def body(s_r, h_r, o_r):
        s = jnp.sum(s_r[...], axis=0, keepdims=True)
        lane = lax.broadcasted_iota(jnp.int32, (1, A_DIM), 1)
        valid = lane < N_META
        z = jnp.where(valid, s * jnp.float32(1.0 / B), -jnp.inf)
        z = z - jnp.max(z)
        ez = jnp.where(valid, jnp.exp(z), 0.0)
        beta = ez / jnp.sum(ez)
        acc = jnp.zeros_like(h_r[0])
        for m in range(N_META):
            bm = jnp.sum(jnp.where(lane == m, beta, 0.0))
            acc = acc + bm * h_r[m]
        o_r[...] = acc

    return pl.pallas_call(
        body,
        grid=(NT,),
        in_specs=[
            pl.BlockSpec((NT, A_DIM), lambda i: (0, 0)),
            pl.BlockSpec((N_META, TB, E), lambda i: (0, i, 0)),
        ],
        out_specs=pl.BlockSpec((TB, E), lambda i: (i, 0)),
        out_shape=jax.ShapeDtypeStruct((B, E), jnp.float32),
    )(s_part, h_all)


def kernel(nodes, x, neigh, W_homo, W_att, b_att, q_att):
    nodes = nodes.astype(jnp.int32)
    neigh = neigh.astype(jnp.int32)
    nb_all = jnp.transpose(neigh, (1, 0, 2)).reshape(N_NODES_C, N_META * DEG)
    nb_all = jnp.pad(nb_all, ((0, 0), (0, NB_W - N_META * DEG)))
    self_feat, agg = _sc_gather_agg(nodes, x, nb_all)
    w_top = W_homo[:, :D, :]
    w_bot = W_homo[:, D:, :]
    h_all, s_part = _tc_encode(
        self_feat, agg, w_top, w_bot, W_att,
        b_att.reshape(1, A_DIM), q_att.reshape(A_DIM, 1),
    )
    return _tc_combine(s_part, h_all)
